# Initial kernel scaffold; baseline (speedup 1.0000x reference)
import jax
import jax.numpy as jnp
from jax.experimental import pallas as pl


def kernel(graph):
    raise NotImplementedError("write your pallas kernel here")


# speedup vs baseline: 17.7220x; 17.7220x over previous
import functools

import jax
import jax.numpy as jnp
from jax import lax
from jax.experimental import pallas as pl
from jax.experimental.pallas import tpu as pltpu
from jax.experimental.pallas import tpu_sc as plsc

N = 1024
B = 32
E = N * (N - 1)
NC = 2
NS = 16
L = 16
NW = NC * NS
ROWS_W = N // NW
IN_CHUNK = ROWS_W * (N - 1)
OUT_CHUNK = ROWS_W * N

_mesh = plsc.VectorSubcoreMesh(core_axis_name="c", subcore_axis_name="s")


@functools.partial(
    pl.kernel,
    out_type=jax.ShapeDtypeStruct((B * N * N,), jnp.float32),
    mesh=_mesh,
    compiler_params=pltpu.CompilerParams(needs_layout_passes=False),
    scratch_types=[
        pltpu.VMEM((IN_CHUNK,), jnp.float32),
        pltpu.VMEM((IN_CHUNK,), jnp.float32),
        pltpu.VMEM((OUT_CHUNK,), jnp.float32),
        pltpu.VMEM((OUT_CHUNK,), jnp.float32),
        pltpu.SemaphoreType.DMA,
        pltpu.SemaphoreType.DMA,
        pltpu.SemaphoreType.DMA,
        pltpu.SemaphoreType.DMA,
    ],
)
def _unflatten_sc(
    g_hbm, out_hbm, in_v0, in_v1, out_v0, out_v1, isem0, isem1, osem0, osem1
):
    wid = lax.axis_index("s") * NC + lax.axis_index("c")
    r0 = wid * ROWS_W
    lane = lax.iota(jnp.int32, 16)

    def in_copy(b, iv, isem):
        g_off = b * E + r0 * (N - 1)
        return pltpu.make_async_copy(g_hbm.at[pl.ds(g_off, IN_CHUNK)], iv, isem)

    def out_copy(b, ov, osem):
        o_off = b * (N * N) + r0 * N
        return pltpu.make_async_copy(ov, out_hbm.at[pl.ds(o_off, OUT_CHUNK)], osem)

    def compute(iv, ov):
        def row_body(i, _):
            r = r0 + i
            ibase = i * (N - 1)
            obase = i * N

            def vreg_body(j, _):
                col = j * L + lane
                idx = ibase + col - (col > r).astype(jnp.int32)
                v = plsc.load_gather(iv, [idx])
                v = jnp.where(col == r, jnp.float32(0.0), v)
                ov[pl.ds(obase + j * L, L)] = v
                return 0

            lax.fori_loop(0, N // L, vreg_body, 0)
            return 0

        lax.fori_loop(0, ROWS_W, row_body, 0)

    slots = ((in_v0, out_v0, isem0, osem0), (in_v1, out_v1, isem1, osem1))

    in_copy(0, in_v0, isem0).start()
    in_copy(1, in_v1, isem1).start()

    def batch_pair(t, _):
        for s in range(2):
            iv, ov, isem, osem = slots[s]
            b = 2 * t + s
            in_copy(b, iv, isem).wait()

            @pl.when(b >= 2)
            def _():
                out_copy(b - 2, ov, osem).wait()

            compute(iv, ov)
            out_copy(b, ov, osem).start()

            @pl.when(b + 2 < B)
            def _():
                in_copy(b + 2, iv, isem).start()

        return 0

    lax.fori_loop(0, B // 2, batch_pair, 0)
    out_copy(B - 2, out_v0, osem0).wait()
    out_copy(B - 1, out_v1, osem1).wait()


def kernel(graph):
    flat = graph.reshape(B * E)
    out = _unflatten_sc(flat)
    return out.reshape(B, N, N)


# speedup vs baseline: 20.0775x; 1.1329x over previous
import functools

import jax
import jax.numpy as jnp
from jax import lax
from jax.experimental import pallas as pl
from jax.experimental.pallas import tpu as pltpu
from jax.experimental.pallas import tpu_sc as plsc

N = 1024
B = 32
E = N * (N - 1)
NC = 2
NS = 16
L = 16
NW = NC * NS
ROWS_W = N // NW
IN_CHUNK = ROWS_W * (N - 1)
OUT_CHUNK = ROWS_W * N

_mesh = plsc.VectorSubcoreMesh(core_axis_name="c", subcore_axis_name="s")


@functools.partial(
    pl.kernel,
    out_type=jax.ShapeDtypeStruct((B * N * N,), jnp.float32),
    mesh=_mesh,
    compiler_params=pltpu.CompilerParams(needs_layout_passes=False),
    scratch_types=[
        pltpu.VMEM((IN_CHUNK,), jnp.float32),
        pltpu.VMEM((IN_CHUNK,), jnp.float32),
        pltpu.VMEM((OUT_CHUNK,), jnp.float32),
        pltpu.VMEM((OUT_CHUNK,), jnp.float32),
        pltpu.SemaphoreType.DMA,
        pltpu.SemaphoreType.DMA,
        pltpu.SemaphoreType.DMA,
        pltpu.SemaphoreType.DMA,
    ],
)
def _unflatten_sc(
    g_hbm, out_hbm, in_v0, in_v1, out_v0, out_v1, isem0, isem1, osem0, osem1
):
    wid = lax.axis_index("s") * NC + lax.axis_index("c")
    r0 = wid * ROWS_W
    lane = lax.iota(jnp.int32, 16)

    def in_copy(b, iv, isem):
        g_off = b * E + r0 * (N - 1)
        return pltpu.make_async_copy(g_hbm.at[pl.ds(g_off, IN_CHUNK)], iv, isem)

    def out_copy(b, ov, osem):
        o_off = b * (N * N) + r0 * N
        return pltpu.make_async_copy(ov, out_hbm.at[pl.ds(o_off, OUT_CHUNK)], osem)

    def compute(iv, ov):
        UNROLL = 8

        def row_body(i, _):
            r = r0 + i
            jr = lax.div(r, L)
            ibase = i * (N - 1)
            obase = i * N

            def vreg_body(jo, _):
                for ju in range(UNROLL):
                    j = jo * UNROLL + ju
                    off = ibase + j * L - (j > jr).astype(jnp.int32)
                    ov[pl.ds(obase + j * L, L)] = iv[pl.ds(off, L)]
                return 0

            lax.fori_loop(0, N // L // UNROLL, vreg_body, 0)

            col = jr * L + lane
            idx = ibase + col - (col > r).astype(jnp.int32)
            v = plsc.load_gather(iv, [idx])
            v = jnp.where(col == r, jnp.float32(0.0), v)
            ov[pl.ds(obase + jr * L, L)] = v
            return 0

        lax.fori_loop(0, ROWS_W, row_body, 0)

    slots = ((in_v0, out_v0, isem0, osem0), (in_v1, out_v1, isem1, osem1))

    in_copy(0, in_v0, isem0).start()
    in_copy(1, in_v1, isem1).start()

    def batch_pair(t, _):
        for s in range(2):
            iv, ov, isem, osem = slots[s]
            b = 2 * t + s
            in_copy(b, iv, isem).wait()

            @pl.when(b >= 2)
            def _():
                out_copy(b - 2, ov, osem).wait()

            compute(iv, ov)
            out_copy(b, ov, osem).start()

            @pl.when(b + 2 < B)
            def _():
                in_copy(b + 2, iv, isem).start()

        return 0

    lax.fori_loop(0, B // 2, batch_pair, 0)
    out_copy(B - 2, out_v0, osem0).wait()
    out_copy(B - 1, out_v1, osem1).wait()


def kernel(graph):
    flat = graph.reshape(B * E)
    out = _unflatten_sc(flat)
    return out.reshape(B, N, N)


# speedup vs baseline: 48.1009x; 2.3958x over previous
import functools

import jax
import jax.numpy as jnp
from jax import lax
from jax.experimental import pallas as pl
from jax.experimental.pallas import tpu as pltpu
from jax.experimental.pallas import tpu_sc as plsc

N = 1024
B = 32
E = N * (N - 1)
NC = 2
NS = 16
L = 16
NW = NC * NS
ROWS_W = N // NW
IN_CHUNK = ROWS_W * (N - 1)

_mesh = plsc.VectorSubcoreMesh(core_axis_name="c", subcore_axis_name="s")


@functools.partial(
    pl.kernel,
    out_type=jax.ShapeDtypeStruct((B, N, N), jnp.float32),
    mesh=_mesh,
    compiler_params=pltpu.CompilerParams(
        needs_layout_passes=False, use_tc_tiling_on_sc=True
    ),
    scratch_types=[
        pltpu.VMEM((IN_CHUNK,), jnp.float32),
        pltpu.VMEM((IN_CHUNK,), jnp.float32),
        pltpu.VMEM((ROWS_W, N), jnp.float32),
        pltpu.VMEM((ROWS_W, N), jnp.float32),
        pltpu.SemaphoreType.DMA,
        pltpu.SemaphoreType.DMA,
        pltpu.SemaphoreType.DMA,
        pltpu.SemaphoreType.DMA,
    ],
)
def _unflatten_sc(
    g_hbm, out_hbm, in_v0, in_v1, out_v0, out_v1, isem0, isem1, osem0, osem1
):
    wid = lax.axis_index("s") * NC + lax.axis_index("c")
    r0 = wid * ROWS_W
    lane = lax.iota(jnp.int32, 16)

    def in_copy(b, iv, isem):
        return pltpu.make_async_copy(
            g_hbm.at[pl.ds(b * E + r0 * (N - 1), IN_CHUNK)], iv, isem
        )

    def out_copy(b, ov, osem):
        return pltpu.make_async_copy(
            ov, out_hbm.at[b, pl.ds(r0, ROWS_W), :], osem
        )

    def compute(iv, ov):
        @functools.partial(plsc.parallel_loop, 0, ROWS_W)
        def row_body(i):
            r = r0 + i
            jr = lax.div(r, L)
            ibase = i * (N - 1)

            col = jr * L + lane
            idx = ibase + col - (col > r).astype(jnp.int32)
            vfix = plsc.load_gather(iv, [idx])
            vfix = jnp.where(col == r, jnp.float32(0.0), vfix)

            for j in range(N // L):
                off = ibase + j * L - (j > jr).astype(jnp.int32)
                v = iv[pl.ds(off, L)]
                sel = jax.lax.broadcast(j == jr, (L,))
                ov[i, pl.ds(j * L, L)] = jnp.where(sel, vfix, v)

    slots = ((in_v0, out_v0, isem0, osem0), (in_v1, out_v1, isem1, osem1))

    in_copy(0, in_v0, isem0).start()
    in_copy(1, in_v1, isem1).start()

    def batch_pair(t, _):
        for s in range(2):
            iv, ov, isem, osem = slots[s]
            b = 2 * t + s
            in_copy(b, iv, isem).wait()

            @pl.when(b >= 2)
            def _():
                out_copy(b - 2, ov, osem).wait()

            compute(iv, ov)
            out_copy(b, ov, osem).start()

            @pl.when(b + 2 < B)
            def _():
                in_copy(b + 2, iv, isem).start()

        return 0

    lax.fori_loop(0, B // 2, batch_pair, 0)
    out_copy(B - 2, out_v0, osem0).wait()
    out_copy(B - 1, out_v1, osem1).wait()


def kernel(graph):
    return _unflatten_sc(graph.reshape(B * E))
